# Initial kernel scaffold; baseline (speedup 1.0000x reference)
import functools

import jax
import jax.numpy as jnp
from jax import lax
from jax.experimental import pallas as pl
from jax.experimental.pallas import tpu as pltpu
from jax.experimental.pallas import tpu_sc as plsc

N_NODES = 10000
N_EDGES = 320000
D_FEAT = 128
D_EDGE = 16
HIDDEN = 64
HPAD = 80
LANES = 16
CH = 128
NC = 2
NS = 16
NW = NC * NS
NCHUNK = N_EDGES // CH
ROWS_PER_TILE = N_NODES // NS


def _prep_ab_body(x_ref, ws_ref, wd_ref, a_ref, b_ref):
    x = x_ref[...]
    a_ref[...] = jnp.dot(x, ws_ref[...], preferred_element_type=jnp.float32)
    b_ref[...] = jnp.dot(x, wd_ref[...], preferred_element_type=jnp.float32)


def _prep_eh_body(ef_ref, we_ref, b1_ref, eh_ref):
    eh_ref[...] = (
        jnp.dot(ef_ref[...], we_ref[...], preferred_element_type=jnp.float32)
        + b1_ref[...]
    )


_sc_mesh = plsc.VectorSubcoreMesh(core_axis_name="c", subcore_axis_name="s")


@functools.partial(
    pl.kernel,
    out_type=jax.ShapeDtypeStruct((NC, N_NODES, HPAD), jnp.float32),
    mesh=_sc_mesh,
    scratch_types=[
        pltpu.VMEM((CH,), jnp.int32),
        pltpu.VMEM((CH,), jnp.int32),
        pltpu.VMEM((CH, HPAD), jnp.float32),
        pltpu.VMEM((CH, HPAD), jnp.float32),
        pltpu.VMEM((CH, HPAD), jnp.float32),
        pltpu.VMEM((CH, HPAD), jnp.float32),
        pltpu.VMEM_SHARED((N_NODES, HPAD), jnp.float32),
        pltpu.SemaphoreType.DMA,
        pltpu.SemaphoreType.DMA,
    ],
)
def _sc_edge(row_hbm, col_hbm, zero_hbm, a_hbm, b_hbm, eh_hbm, out_hbm,
             rowi, coli, av, bv, ev, hv, acc, g1sem, g2sem):
    cid = lax.axis_index("c")
    sid = lax.axis_index("s")
    wid = sid * NC + cid

    pltpu.sync_copy(zero_hbm, acc.at[pl.ds(sid * ROWS_PER_TILE, ROWS_PER_TILE)])
    plsc.subcore_barrier()

    nmine = (NCHUNK - wid + NW - 1) // NW

    def chunk_body(i, carry):
        base = (wid + i * NW) * CH
        pltpu.sync_copy(row_hbm.at[pl.ds(base, CH)], rowi)
        pltpu.sync_copy(col_hbm.at[pl.ds(base, CH)], coli)
        cp1 = pltpu.async_copy(a_hbm.at[rowi], av, g1sem)
        cp2 = pltpu.async_copy(b_hbm.at[coli], bv, g2sem)
        pltpu.sync_copy(eh_hbm.at[pl.ds(base, CH)], ev)
        cp1.wait()
        cp2.wait()

        def row_body(r, c2):
            for j in range(HPAD // LANES):
                s = pl.ds(j * LANES, LANES)
                hv[r, s] = jnp.maximum(av[r, s] + bv[r, s] + ev[r, s], 0.0)
            return c2

        lax.fori_loop(0, CH, row_body, 0)
        pltpu.sync_copy(hv, acc.at[coli], add=True)
        return carry

    lax.fori_loop(0, nmine, chunk_body, 0)
    plsc.subcore_barrier()
    pltpu.sync_copy(
        acc.at[pl.ds(sid * ROWS_PER_TILE, ROWS_PER_TILE)],
        out_hbm.at[cid, pl.ds(sid * ROWS_PER_TILE, ROWS_PER_TILE)],
    )


def _post_body(a0_ref, a1_ref, x_ref, w2_ref, ua_ref, ub_ref, ub1_ref,
               uw2_ref, ub2_ref, o_ref):
    hsum = a0_ref[...] + a1_ref[...]
    agg = jnp.dot(hsum, w2_ref[...], preferred_element_type=jnp.float32)
    x = x_ref[...]
    u = jnp.maximum(
        jnp.dot(x, ua_ref[...], preferred_element_type=jnp.float32)
        + jnp.dot(agg, ub_ref[...], preferred_element_type=jnp.float32)
        + ub1_ref[...],
        0.0,
    )
    o_ref[...] = (
        jnp.dot(u, uw2_ref[...], preferred_element_type=jnp.float32)
        + ub2_ref[...]
    )


def kernel(nodes, edge_indices, edge_features, msg_w1, msg_b1, msg_w2,
           msg_b2, upd_w1, upd_b1, upd_w2, upd_b2):
    ei = edge_indices.astype(jnp.int32)
    row, col = ei[0], ei[1]

    padw = jnp.zeros((D_FEAT, HPAD - HIDDEN), jnp.float32)
    w1s_pad = jnp.concatenate([msg_w1[:D_FEAT], padw], axis=1)
    w1d_pad = jnp.concatenate([msg_w1[D_FEAT:2 * D_FEAT], padw], axis=1)
    w1e_pad = jnp.concatenate(
        [msg_w1[2 * D_FEAT:], jnp.zeros((D_EDGE, HPAD - HIDDEN), jnp.float32)],
        axis=1)
    b1_pad = jnp.concatenate(
        [msg_b1, jnp.ones((1,), jnp.float32),
         jnp.zeros((HPAD - HIDDEN - 1,), jnp.float32)])[None, :]

    rb = N_NODES // 5
    a_tab, b_tab = pl.pallas_call(
        _prep_ab_body,
        grid=(5,),
        in_specs=[
            pl.BlockSpec((rb, D_FEAT), lambda i: (i, 0)),
            pl.BlockSpec((D_FEAT, HPAD), lambda i: (0, 0)),
            pl.BlockSpec((D_FEAT, HPAD), lambda i: (0, 0)),
        ],
        out_specs=[
            pl.BlockSpec((rb, HPAD), lambda i: (i, 0)),
            pl.BlockSpec((rb, HPAD), lambda i: (i, 0)),
        ],
        out_shape=[
            jax.ShapeDtypeStruct((N_NODES, HPAD), jnp.float32),
            jax.ShapeDtypeStruct((N_NODES, HPAD), jnp.float32),
        ],
    )(nodes, w1s_pad, w1d_pad)

    eb = N_EDGES // 32
    eh = pl.pallas_call(
        _prep_eh_body,
        grid=(32,),
        in_specs=[
            pl.BlockSpec((eb, D_EDGE), lambda i: (i, 0)),
            pl.BlockSpec((D_EDGE, HPAD), lambda i: (0, 0)),
            pl.BlockSpec((1, HPAD), lambda i: (0, 0)),
        ],
        out_specs=pl.BlockSpec((eb, HPAD), lambda i: (i, 0)),
        out_shape=jax.ShapeDtypeStruct((N_EDGES, HPAD), jnp.float32),
    )(edge_features, w1e_pad, b1_pad)

    zero_blk = jnp.zeros((ROWS_PER_TILE, HPAD), jnp.float32)
    acc = _sc_edge(row, col, zero_blk, a_tab, b_tab, eh)

    w2_pad = jnp.concatenate(
        [msg_w2, msg_b2[None, :],
         jnp.zeros((HPAD - HIDDEN - 1, D_FEAT), jnp.float32)], axis=0)

    out = pl.pallas_call(
        _post_body,
        grid=(5,),
        in_specs=[
            pl.BlockSpec((rb, HPAD), lambda i: (i, 0)),
            pl.BlockSpec((rb, HPAD), lambda i: (i, 0)),
            pl.BlockSpec((rb, D_FEAT), lambda i: (i, 0)),
            pl.BlockSpec((HPAD, D_FEAT), lambda i: (0, 0)),
            pl.BlockSpec((D_FEAT, HIDDEN), lambda i: (0, 0)),
            pl.BlockSpec((D_FEAT, HIDDEN), lambda i: (0, 0)),
            pl.BlockSpec((1, HIDDEN), lambda i: (0, 0)),
            pl.BlockSpec((HIDDEN, D_FEAT), lambda i: (0, 0)),
            pl.BlockSpec((1, D_FEAT), lambda i: (0, 0)),
        ],
        out_specs=pl.BlockSpec((rb, D_FEAT), lambda i: (i, 0)),
        out_shape=jax.ShapeDtypeStruct((N_NODES, D_FEAT), jnp.float32),
    )(acc[0], acc[1], nodes, w2_pad, upd_w1[:D_FEAT], upd_w1[D_FEAT:],
      upd_b1[None, :], upd_w2, upd_b2[None, :])
    return out


# speedup vs baseline: 1.6455x; 1.6455x over previous
import functools

import jax
import jax.numpy as jnp
from jax import lax
from jax.experimental import pallas as pl
from jax.experimental.pallas import tpu as pltpu
from jax.experimental.pallas import tpu_sc as plsc

N_NODES = 10000
N_EDGES = 320000
D_FEAT = 128
D_EDGE = 16
HIDDEN = 64
HPAD = 128
APAD = 64
LANES = 16
CH = 128
NC = 2
NS = 16
NW = NC * NS
NCHUNK = N_EDGES // CH
N_PAD = 10240
ROWS_PER_TILE = N_PAD // NS


def _prep_ab_body(x_ref, ws_ref, wd_ref, a_ref, b_ref):
    x = x_ref[...]
    a_ref[...] = jnp.dot(x, ws_ref[...], preferred_element_type=jnp.float32)
    b_ref[...] = jnp.dot(x, wd_ref[...], preferred_element_type=jnp.float32)


def _prep_eh_body(ef_ref, we_ref, b1_ref, eh_ref):
    eh_ref[...] = (
        jnp.dot(ef_ref[...], we_ref[...], preferred_element_type=jnp.float32)
        + b1_ref[...]
    )


EPW = N_EDGES // NW
CPW_MAX = (NCHUNK + NW - 1) // NW
IDXW = CPW_MAX * CH


@functools.cache
def _make_sc_edge():
  mesh = plsc.VectorSubcoreMesh(
      core_axis_name="c", subcore_axis_name="s", num_cores=NC, num_subcores=NS)

  @functools.partial(
      pl.kernel,
      out_type=(jax.ShapeDtypeStruct((N_EDGES, HPAD), jnp.float32),
                jax.ShapeDtypeStruct((N_EDGES, HPAD), jnp.float32)),
      mesh=mesh,
      scratch_types=[
          pltpu.VMEM((CH,), jnp.int32),
          pltpu.VMEM((CH,), jnp.int32),
          pltpu.VMEM((CH, HPAD), jnp.float32),
          pltpu.VMEM((CH, HPAD), jnp.float32),
          pltpu.SemaphoreType.DMA,
      ],
  )
  def _sc_edge(row_hbm, col_hbm, a_hbm, b_hbm, srcg_hbm, dstg_hbm,
               rowi, coli, av, bv, dsem):
    cid = lax.axis_index("c")
    sid = lax.axis_index("s")
    wid = sid * NC + cid

    nextra = NCHUNK % NW
    start_chunk = wid * (NCHUNK // NW) + jnp.minimum(wid, nextra)
    nmine = jnp.where(wid < nextra, NCHUNK // NW + 1, NCHUNK // NW)
    ebase = start_chunk * CH

    @pl.loop(0, nmine)
    def src_loop(i):
      pltpu.sync_copy(row_hbm.at[pl.ds(ebase + i * CH, CH)], rowi)
      pltpu.async_copy(a_hbm.at[rowi], av, dsem).wait()
      pltpu.sync_copy(av, srcg_hbm.at[pl.ds(ebase + i * CH, CH)])

    @pl.loop(0, nmine)
    def dst_loop(i):
      pltpu.sync_copy(col_hbm.at[pl.ds(ebase + i * CH, CH)], coli)
      pltpu.async_copy(b_hbm.at[coli], bv, dsem).wait()
      pltpu.sync_copy(bv, dstg_hbm.at[pl.ds(ebase + i * CH, CH)])

  return _sc_edge


def _edge_relu_body(sg_ref, dg_ref, eh_ref, h_ref):
    h_ref[...] = jnp.maximum(
        sg_ref[...] + dg_ref[...] + eh_ref[...], 0.0)[:, :HIDDEN]


def _post_body(a0_ref, d0_ref, x_ref, w2_ref, b2_ref,
               ua_ref, ub_ref, ub1_ref, uw2_ref, ub2_ref, o_ref):
    agg = (jnp.dot(a0_ref[...], w2_ref[...], preferred_element_type=jnp.float32)
           + d0_ref[...] * b2_ref[...])
    x = x_ref[...]
    u = jnp.maximum(
        jnp.dot(x, ua_ref[...], preferred_element_type=jnp.float32)
        + jnp.dot(agg, ub_ref[...], preferred_element_type=jnp.float32)
        + ub1_ref[...],
        0.0,
    )
    o_ref[...] = (
        jnp.dot(u, uw2_ref[...], preferred_element_type=jnp.float32)
        + ub2_ref[...]
    )


def kernel(nodes, edge_indices, edge_features, msg_w1, msg_b1, msg_w2,
           msg_b2, upd_w1, upd_b1, upd_w2, upd_b2):
    ei = edge_indices.astype(jnp.int32)
    row = jnp.concatenate([ei[0], jnp.zeros((CH,), jnp.int32)])
    col = jnp.concatenate([ei[1], jnp.zeros((CH,), jnp.int32)])

    padw = jnp.zeros((D_FEAT, HPAD - HIDDEN), jnp.float32)
    w1s_pad = jnp.concatenate([msg_w1[:D_FEAT], padw], axis=1)
    w1d_pad = jnp.concatenate([msg_w1[D_FEAT:2 * D_FEAT], padw], axis=1)
    w1e_pad = jnp.concatenate(
        [msg_w1[2 * D_FEAT:], jnp.zeros((D_EDGE, HPAD - HIDDEN), jnp.float32)],
        axis=1)
    b1_pad = jnp.concatenate(
        [msg_b1, jnp.zeros((HPAD - HIDDEN,), jnp.float32)])[None, :]

    rb = N_NODES // 5
    a_tab, b_tab = pl.pallas_call(
        _prep_ab_body,
        grid=(5,),
        in_specs=[
            pl.BlockSpec((rb, D_FEAT), lambda i: (i, 0)),
            pl.BlockSpec((D_FEAT, HPAD), lambda i: (0, 0)),
            pl.BlockSpec((D_FEAT, HPAD), lambda i: (0, 0)),
        ],
        out_specs=[
            pl.BlockSpec((rb, HPAD), lambda i: (i, 0)),
            pl.BlockSpec((rb, HPAD), lambda i: (i, 0)),
        ],
        out_shape=[
            jax.ShapeDtypeStruct((N_NODES, HPAD), jnp.float32),
            jax.ShapeDtypeStruct((N_NODES, HPAD), jnp.float32),
        ],
    )(nodes, w1s_pad, w1d_pad)

    eb = N_EDGES // 32
    eh = pl.pallas_call(
        _prep_eh_body,
        grid=(32,),
        in_specs=[
            pl.BlockSpec((eb, D_EDGE), lambda i: (i, 0)),
            pl.BlockSpec((D_EDGE, HPAD), lambda i: (0, 0)),
            pl.BlockSpec((1, HPAD), lambda i: (0, 0)),
        ],
        out_specs=pl.BlockSpec((eb, HPAD), lambda i: (i, 0)),
        out_shape=jax.ShapeDtypeStruct((N_EDGES, HPAD), jnp.float32),
    )(edge_features, w1e_pad, b1_pad)

    srcg, dstg = _make_sc_edge()(row, col, a_tab, b_tab)

    h = pl.pallas_call(
        _edge_relu_body,
        grid=(32,),
        in_specs=[
            pl.BlockSpec((eb, HPAD), lambda i: (i, 0)),
            pl.BlockSpec((eb, HPAD), lambda i: (i, 0)),
            pl.BlockSpec((eb, HPAD), lambda i: (i, 0)),
        ],
        out_specs=pl.BlockSpec((eb, HIDDEN), lambda i: (i, 0)),
        out_shape=jax.ShapeDtypeStruct((N_EDGES, HIDDEN), jnp.float32),
    )(srcg, dstg, eh)

    col_e = ei[1]
    hsum = jnp.zeros((N_NODES, HIDDEN), jnp.float32).at[col_e].add(h)
    deg = jnp.zeros((N_NODES, 1), jnp.float32).at[col_e, 0].add(1.0)

    out = pl.pallas_call(
        _post_body,
        grid=(5,),
        in_specs=[
            pl.BlockSpec((rb, HIDDEN), lambda i: (i, 0)),
            pl.BlockSpec((rb, 1), lambda i: (i, 0)),
            pl.BlockSpec((rb, D_FEAT), lambda i: (i, 0)),
            pl.BlockSpec((HIDDEN, D_FEAT), lambda i: (0, 0)),
            pl.BlockSpec((1, D_FEAT), lambda i: (0, 0)),
            pl.BlockSpec((D_FEAT, HIDDEN), lambda i: (0, 0)),
            pl.BlockSpec((D_FEAT, HIDDEN), lambda i: (0, 0)),
            pl.BlockSpec((1, HIDDEN), lambda i: (0, 0)),
            pl.BlockSpec((HIDDEN, D_FEAT), lambda i: (0, 0)),
            pl.BlockSpec((1, D_FEAT), lambda i: (0, 0)),
        ],
        out_specs=pl.BlockSpec((rb, D_FEAT), lambda i: (i, 0)),
        out_shape=jax.ShapeDtypeStruct((N_NODES, D_FEAT), jnp.float32),
    )(hsum, deg, nodes, msg_w2, msg_b2[None, :],
      upd_w1[:D_FEAT], upd_w1[D_FEAT:],
      upd_b1[None, :], upd_w2, upd_b2[None, :])
    return out
